# baseline (speedup 1.0000x reference)
import functools

import jax
import jax.numpy as jnp
from jax import lax
from jax.experimental import pallas as pl
from jax.experimental.pallas import tpu as pltpu
from jax.experimental.pallas import tpu_sc as plsc

EMBED = 32
LANES = 16
CHUNK = 128


def _sc_body(nc, bpw, user_h, course_h, tab_u_h, tab_c_h, wb_h, out_h,
             idx_u, idx_c, rows_u, rows_c, out_v, wb_v, sem):
    wid = lax.axis_index("s") * nc + lax.axis_index("c")
    base = wid * bpw
    nch = bpw // CHUNK

    pltpu.sync_copy(wb_h, wb_v)
    for j in range(nch):
        pltpu.sync_copy(user_h.at[pl.ds(base + j * CHUNK, CHUNK)], idx_u.at[j])
        pltpu.sync_copy(course_h.at[pl.ds(base + j * CHUNK, CHUNK)], idx_c.at[j])

    copies = []
    for j in range(nch):
        copies.append(pltpu.async_copy(
            tab_u_h.at[idx_u.at[j]], rows_u.at[pl.ds(j * CHUNK, CHUNK)], sem))
        copies.append(pltpu.async_copy(
            tab_c_h.at[idx_c.at[j]], rows_c.at[pl.ds(j * CHUNK, CHUNK)], sem))
    for c in copies:
        c.wait()

    w_ul = wb_v[0, :]
    w_uh = wb_v[1, :]
    w_cl = wb_v[2, :]
    w_ch = wb_v[3, :]
    b_vec = wb_v[4, :]
    lane = lax.iota(jnp.int32, LANES)

    def grp(g, carry):
        acc = b_vec
        for u in range(LANES):
            i = g * LANES + u
            t = (rows_u[i, pl.ds(0, LANES)] * w_ul
                 + rows_u[i, pl.ds(LANES, LANES)] * w_uh
                 + rows_c[i, pl.ds(0, LANES)] * w_cl
                 + rows_c[i, pl.ds(LANES, LANES)] * w_ch)
            s = jnp.sum(t)
            acc = jnp.where(lane == u, acc + s, acc)
        out_v[pl.ds(g * LANES, LANES)] = acc
        return carry

    lax.fori_loop(0, bpw // LANES, grp, 0)

    pltpu.sync_copy(out_v, out_h.at[pl.ds(base, bpw)])


@jax.jit
def _run(user, course, user_table, course_table, wb):
    batch = user.shape[0]
    info = plsc.get_sparse_core_info()
    nc, ns = info.num_cores, info.num_subcores
    nw = nc * ns
    bpw = batch // nw

    mesh = plsc.VectorSubcoreMesh(core_axis_name="c", subcore_axis_name="s")
    fn = pl.kernel(
        functools.partial(_sc_body, nc, bpw),
        out_type=jax.ShapeDtypeStruct((batch,), jnp.float32),
        mesh=mesh,
        compiler_params=pltpu.CompilerParams(
            needs_layout_passes=False, use_tc_tiling_on_sc=False),
        scratch_types=[
            pltpu.VMEM((bpw // CHUNK, CHUNK), jnp.int32),
            pltpu.VMEM((bpw // CHUNK, CHUNK), jnp.int32),
            pltpu.VMEM((bpw, EMBED), jnp.float32),
            pltpu.VMEM((bpw, EMBED), jnp.float32),
            pltpu.VMEM((bpw,), jnp.float32),
            pltpu.VMEM((5, LANES), jnp.float32),
            pltpu.SemaphoreType.DMA,
        ],
    )
    return fn(user, course, user_table, course_table, wb)


def kernel(user, course, user_table, course_table, W, b):
    w4 = W.reshape(4, LANES)
    b16 = jnp.broadcast_to(b, (1, LANES)).astype(jnp.float32)
    wb = jnp.concatenate([w4, b16], axis=0)
    out = _run(user, course, user_table, course_table, wb)
    return out.reshape(-1, 1)


# speedup vs baseline: 7.5084x; 7.5084x over previous
import functools

import jax
import jax.numpy as jnp
from jax import lax
from jax.experimental import pallas as pl
from jax.experimental.pallas import tpu as pltpu
from jax.experimental.pallas import tpu_sc as plsc

EMBED = 32
LANES = 16
CHUNK = 128
PROJ_BLK = 65536


def _proj_body(w_ref, tab_ref, out_ref):
    out_ref[...] = jnp.dot(
        w_ref[...], tab_ref[...], preferred_element_type=jnp.float32)[0]


def _tc_project(w_row, tab_t):
    n = tab_t.shape[1]
    grid = pl.cdiv(n, PROJ_BLK)
    return pl.pallas_call(
        _proj_body,
        grid=(grid,),
        in_specs=[
            pl.BlockSpec((1, EMBED), lambda i: (0, 0)),
            pl.BlockSpec((EMBED, PROJ_BLK), lambda i: (0, i)),
        ],
        out_specs=pl.BlockSpec((PROJ_BLK,), lambda i: (i,)),
        out_shape=jax.ShapeDtypeStruct((n,), jnp.float32),
    )(w_row, tab_t)


def _sc_body(nc, bpw, user_h, course_h, pu_h, pc_h, bv_h, out_h,
             idx_u, idx_c, g_u, g_c, bv_v, out_v, sem):
    wid = lax.axis_index("s") * nc + lax.axis_index("c")
    base = wid * bpw
    nch = bpw // CHUNK

    pltpu.sync_copy(bv_h, bv_v)
    for j in range(nch):
        pltpu.sync_copy(user_h.at[pl.ds(base + j * CHUNK, CHUNK)], idx_u.at[j])
        pltpu.sync_copy(course_h.at[pl.ds(base + j * CHUNK, CHUNK)], idx_c.at[j])

    copies = []
    for j in range(nch):
        copies.append(pltpu.async_copy(pu_h.at[idx_u.at[j]], g_u.at[j], sem))
        copies.append(pltpu.async_copy(pc_h.at[idx_c.at[j]], g_c.at[j], sem))
    for c in copies:
        c.wait()

    b_vec = bv_v[...]
    for j in range(nch):
        for m in range(0, CHUNK, LANES):
            t = g_u[j, pl.ds(m, LANES)] + g_c[j, pl.ds(m, LANES)] + b_vec
            out_v[pl.ds(j * CHUNK + m, LANES)] = t

    pltpu.sync_copy(out_v, out_h.at[pl.ds(base, bpw)])


def _sc_gather_add(user, course, proj_u, proj_c, b_vec):
    batch = user.shape[0]
    info = plsc.get_sparse_core_info()
    nc, ns = info.num_cores, info.num_subcores
    bpw = batch // (nc * ns)

    mesh = plsc.VectorSubcoreMesh(core_axis_name="c", subcore_axis_name="s")
    fn = pl.kernel(
        functools.partial(_sc_body, nc, bpw),
        out_type=jax.ShapeDtypeStruct((batch,), jnp.float32),
        mesh=mesh,
        compiler_params=pltpu.CompilerParams(
            needs_layout_passes=False, use_tc_tiling_on_sc=False),
        scratch_types=[
            pltpu.VMEM((bpw // CHUNK, CHUNK), jnp.int32),
            pltpu.VMEM((bpw // CHUNK, CHUNK), jnp.int32),
            pltpu.VMEM((bpw // CHUNK, CHUNK), jnp.float32),
            pltpu.VMEM((bpw // CHUNK, CHUNK), jnp.float32),
            pltpu.VMEM((LANES,), jnp.float32),
            pltpu.VMEM((bpw,), jnp.float32),
            pltpu.SemaphoreType.DMA,
        ],
    )
    return fn(user, course, proj_u, proj_c, b_vec)


@jax.jit
def _run(user, course, user_table, course_table, W, b):
    w_u = W[:, :EMBED]
    w_c = W[:, EMBED:]
    proj_u = _tc_project(w_u, user_table.T)
    proj_c = _tc_project(w_c, course_table.T)
    b_vec = jnp.broadcast_to(b, (LANES,)).astype(jnp.float32)
    return _sc_gather_add(user, course, proj_u, proj_c, b_vec)


def kernel(user, course, user_table, course_table, W, b):
    out = _run(user, course, user_table, course_table, W, b)
    return out.reshape(-1, 1)
